# baseline (speedup 1.0000x reference)
import functools

import jax
import jax.numpy as jnp
from jax import lax
from jax.experimental import pallas as pl
from jax.experimental.pallas import tpu as pltpu
from jax.experimental.pallas import tpu_sc as plsc

_B = 16384
_D = 64
_NW = 32
_BPW = _B // _NW
_CH = 32
_NCH = _BPW // _CH
_L = 16


@functools.partial(
    pl.kernel,
    out_type=jax.ShapeDtypeStruct((_NW, _L), jnp.float32),
    mesh=plsc.VectorSubcoreMesh(core_axis_name="c", subcore_axis_name="s"),
    scratch_types=[
        pltpu.VMEM((_NCH, _CH), jnp.int32),
        pltpu.VMEM((_BPW,), jnp.int32),
        pltpu.VMEM((2, _CH, 2 * _D), jnp.float32),
        pltpu.VMEM((_BPW, _D), jnp.float32),
        pltpu.VMEM((_L,), jnp.float32),
        pltpu.SemaphoreType.DMA,
        pltpu.SemaphoreType.DMA,
    ],
)
def _center_loss_sc(feat_hbm, hi_hbm, lo_hbm, cent_hbm, out_hbm,
                    hi_v, lo_v, rows_v, feat_v, acc_v, sem0, sem1):
    wid = lax.axis_index("s") * 2 + lax.axis_index("c")
    sems = (sem0, sem1)

    pltpu.sync_copy(hi_hbm.at[wid], hi_v)
    pltpu.sync_copy(lo_hbm.at[wid], lo_v)

    def copy_op(j, b):
        return pltpu.make_async_copy(
            cent_hbm.at[hi_v.at[j]], rows_v.at[b], sems[b])

    copy_op(0, 0).start()
    copy_op(1, 1).start()
    pltpu.sync_copy(feat_hbm.at[wid], feat_v)

    def pair_body(j2, acc):
        for b in range(2):
            j = j2 * 2 + b
            copy_op(j, b).wait()
            off_vec = lo_v[pl.ds(j * _CH, _CH)]
            a = jnp.zeros((_L,), jnp.float32)
            for k in range(_CH):
                off = off_vec[k]
                for ci in range(_D // _L):
                    f = feat_v[j * _CH + k, pl.ds(ci * _L, _L)]
                    c = rows_v[b, k, pl.ds(off + ci * _L, _L)]
                    d = f - c
                    a = a + d * d
            acc = acc + a

            @pl.when(j + 2 < _NCH)
            def _():
                copy_op(j + 2, b).start()
        return acc

    acc = lax.fori_loop(0, _NCH // 2, pair_body, jnp.zeros((_L,), jnp.float32))
    acc_v[...] = acc
    pltpu.sync_copy(acc_v, out_hbm.at[wid])


def kernel(features, labels, centers):
    batch = features.shape[0]
    lab = labels.astype(jnp.int32)
    hi = (lab >> 1).reshape(_NW, _NCH, _CH)
    lo = ((lab & 1) * _D).reshape(_NW, _BPW)
    feat_r = features.reshape(_NW, _BPW, _D)
    cent_r = centers.reshape(centers.shape[0] // 2, 2 * _D)
    partials = _center_loss_sc(feat_r, hi, lo, cent_r)
    return jnp.sum(partials) / 2.0 / batch


# speedup vs baseline: 1.6541x; 1.6541x over previous
import functools

import jax
import jax.numpy as jnp
from jax import lax
from jax.experimental import pallas as pl
from jax.experimental.pallas import tpu as pltpu
from jax.experimental.pallas import tpu_sc as plsc

_B = 16384
_D = 64
_NW = 32
_BPW = _B // _NW
_CH = 16
_NCH = _BPW // _CH
_L = 16


@functools.partial(
    pl.kernel,
    out_type=jax.ShapeDtypeStruct((_NW, _L), jnp.float32),
    mesh=plsc.VectorSubcoreMesh(core_axis_name="c", subcore_axis_name="s"),
    scratch_types=[
        pltpu.VMEM((_BPW,), jnp.int32),
        pltpu.VMEM((2, _CH, _D), jnp.float32),
        pltpu.VMEM((_BPW, _D), jnp.float32),
        pltpu.VMEM((_L,), jnp.float32),
        pltpu.SemaphoreType.DMA,
        pltpu.SemaphoreType.DMA,
    ],
)
def _center_loss_sc(feat_hbm, lab_hbm, cent_hbm, out_hbm,
                    lab_v, rows_v, feat_v, acc_v, sem0, sem1):
    wid = lax.axis_index("s") * 2 + lax.axis_index("c")
    sems = (sem0, sem1)

    pltpu.sync_copy(lab_hbm.at[wid], lab_v)

    def fire(j, b):
        idx_vec = lab_v[pl.ds(j * _CH, _CH)]
        for k in range(_CH):
            pltpu.make_async_copy(
                cent_hbm.at[idx_vec[k]], rows_v.at[b, k], sems[b]).start()

    def drain(b):
        for k in range(_CH):
            pltpu.make_async_copy(
                cent_hbm.at[0], rows_v.at[b, k], sems[b]).wait()

    fire(0, 0)
    pltpu.sync_copy(feat_hbm.at[wid], feat_v)

    def body(j2, acc):
        for b in range(2):
            j = j2 * 2 + b

            @pl.when(j + 1 < _NCH)
            def _(j=j, b=b):
                fire(j + 1, (b + 1) % 2)

            drain(b)
            a = jnp.zeros((_L,), jnp.float32)
            for k in range(_CH):
                for ci in range(_D // _L):
                    f = feat_v[j * _CH + k, pl.ds(ci * _L, _L)]
                    c = rows_v[b, k, pl.ds(ci * _L, _L)]
                    d = f - c
                    a = a + d * d
            acc = acc + a
        return acc

    acc = lax.fori_loop(0, _NCH // 2, body, jnp.zeros((_L,), jnp.float32))
    acc_v[...] = acc
    pltpu.sync_copy(acc_v, out_hbm.at[wid])


def kernel(features, labels, centers):
    batch = features.shape[0]
    lab = labels.astype(jnp.int32).reshape(_NW, _BPW)
    feat_r = features.reshape(_NW, _BPW, _D)
    partials = _center_loss_sc(feat_r, lab, centers)
    return jnp.sum(partials) / 2.0 / batch


# speedup vs baseline: 1.6637x; 1.0058x over previous
import functools

import jax
import jax.numpy as jnp
from jax import lax
from jax.experimental import pallas as pl
from jax.experimental.pallas import tpu as pltpu
from jax.experimental.pallas import tpu_sc as plsc

_B = 16384
_D = 64
_NW = 32
_BPW = _B // _NW
_CH = 16
_NCH = _BPW // _CH
_L = 16


@functools.partial(
    pl.kernel,
    out_type=jax.ShapeDtypeStruct((_NW, _L), jnp.float32),
    mesh=plsc.VectorSubcoreMesh(core_axis_name="c", subcore_axis_name="s"),
    scratch_types=[
        pltpu.VMEM((_BPW,), jnp.int32),
        pltpu.VMEM((2, _CH, _D), jnp.float32),
        pltpu.VMEM((2, _CH, _D), jnp.float32),
        pltpu.VMEM((_L,), jnp.float32),
        pltpu.SemaphoreType.DMA,
        pltpu.SemaphoreType.DMA,
    ],
)
def _center_loss_sc(feat_hbm, lab_hbm, cent_hbm, out_hbm,
                    lab_v, crows_v, frows_v, acc_v, sem0, sem1):
    wid = lax.axis_index("s") * 2 + lax.axis_index("c")
    sems = (sem0, sem1)

    pltpu.sync_copy(lab_hbm.at[pl.ds(wid * _BPW, _BPW)], lab_v)

    def fire(j, b):
        idx_vec = lab_v[pl.ds(j * _CH, _CH)]
        base = wid * _BPW + j * _CH
        for k in range(_CH):
            pltpu.make_async_copy(
                cent_hbm.at[idx_vec[k]], crows_v.at[b, k], sems[b]).start()
            pltpu.make_async_copy(
                feat_hbm.at[base + k], frows_v.at[b, k], sems[b]).start()

    def drain(b):
        for k in range(_CH):
            pltpu.make_async_copy(
                cent_hbm.at[0], crows_v.at[b, k], sems[b]).wait()
            pltpu.make_async_copy(
                cent_hbm.at[0], frows_v.at[b, k], sems[b]).wait()

    fire(0, 0)

    def body(j2, acc):
        for b in range(2):
            j = j2 * 2 + b

            @pl.when(j + 1 < _NCH)
            def _(j=j, b=b):
                fire(j + 1, (b + 1) % 2)

            drain(b)
            a = jnp.zeros((_L,), jnp.float32)
            for k in range(_CH):
                for ci in range(_D // _L):
                    f = frows_v[b, k, pl.ds(ci * _L, _L)]
                    c = crows_v[b, k, pl.ds(ci * _L, _L)]
                    d = f - c
                    a = a + d * d
            acc = acc + a
        return acc

    acc = lax.fori_loop(0, _NCH // 2, body, jnp.zeros((_L,), jnp.float32))
    acc_v[...] = acc
    pltpu.sync_copy(acc_v, out_hbm.at[wid])


def kernel(features, labels, centers):
    batch = features.shape[0]
    lab = labels.astype(jnp.int32)
    partials = _center_loss_sc(features, lab, centers)
    return jnp.sum(partials) / 2.0 / batch
